# baseline (speedup 1.0000x reference)
import functools

import jax
import jax.numpy as jnp
from jax import lax
from jax.experimental import pallas as pl
from jax.experimental.pallas import tpu as pltpu
from jax.experimental.pallas import tpu_sc as plsc

N = 10000
E = 320000
DIN = 128
DH = 128
DOUT = 64

NC = 2
NS = 16
NW = NC * NS
_EPT = E // NW
_CHUNK = 100
_NCH = _EPT // _CHUNK
_NR = 624
_NTAIL = N - NS * _NR


def _mesh():
    return plsc.VectorSubcoreMesh(core_axis_name="c", subcore_axis_name="s")


def _sc_hist(dst3, ones, zrows):
    @functools.partial(
        pl.kernel,
        mesh=_mesh(),
        out_type=jax.ShapeDtypeStruct((NC, N, DH), jnp.float32),
        scratch_types=[
            pltpu.VMEM((_NCH, _CHUNK), jnp.int32),
            pltpu.VMEM((_CHUNK, DH), jnp.float32),
            pltpu.VMEM_SHARED((N, DH), jnp.float32),
        ],
    )
    def k(dst_hbm, ones_hbm, z_hbm, out_hbm, di_v, ones_v, acc_sh):
        cid = lax.axis_index("c")
        sid = lax.axis_index("s")
        wid = cid * NS + sid
        r0 = sid * _NR
        pltpu.sync_copy(z_hbm, acc_sh.at[pl.ds(r0, _NR)])

        @pl.when(sid == 0)
        def _():
            pltpu.sync_copy(z_hbm.at[pl.ds(0, _NTAIL)],
                            acc_sh.at[pl.ds(NS * _NR, _NTAIL)])

        pltpu.sync_copy(ones_hbm, ones_v)
        pltpu.sync_copy(dst_hbm.at[wid], di_v)
        plsc.subcore_barrier()

        @pl.loop(0, _NCH)
        def _(j):
            pltpu.sync_copy(ones_v, acc_sh.at[di_v.at[j]], add=True)

        plsc.subcore_barrier()
        pltpu.sync_copy(acc_sh.at[pl.ds(r0, _NR)], out_hbm.at[cid, pl.ds(r0, _NR)])

        @pl.when(sid == 0)
        def _():
            pltpu.sync_copy(acc_sh.at[pl.ds(NS * _NR, _NTAIL)],
                            out_hbm.at[cid, pl.ds(NS * _NR, _NTAIL)])

    return k(dst3, ones, zrows)


def _sc_conv(hp, ei4, zrows, d):
    @functools.partial(
        pl.kernel,
        mesh=_mesh(),
        out_type=jax.ShapeDtypeStruct((NC, N, d), jnp.float32),
        scratch_types=[
            pltpu.VMEM((2, _CHUNK), jnp.int32),
            pltpu.VMEM((2, _CHUNK), jnp.int32),
            pltpu.VMEM((2, _CHUNK), jnp.int32),
            pltpu.VMEM((_CHUNK, d), jnp.float32),
            pltpu.VMEM((_CHUNK, d), jnp.float32),
            pltpu.VMEM((_CHUNK, d), jnp.float32),
            pltpu.VMEM_SHARED((N, d), jnp.float32),
            pltpu.SemaphoreType.DMA,
            pltpu.SemaphoreType.DMA,
            pltpu.SemaphoreType.DMA,
            pltpu.SemaphoreType.DMA,
            pltpu.SemaphoreType.DMA,
            pltpu.SemaphoreType.DMA,
        ],
    )
    def k(hp_hbm, ei_hbm, z_hbm, out_hbm, ei_a, ei_b, ei_c,
          rows_a, rows_b, rows_c, acc_sh,
          gsem_a, gsem_b, gsem_c, isem_a, isem_b, isem_c):
        cid = lax.axis_index("c")
        sid = lax.axis_index("s")
        wid = cid * NS + sid
        r0 = sid * _NR
        pltpu.sync_copy(z_hbm, acc_sh.at[pl.ds(r0, _NR)])

        @pl.when(sid == 0)
        def _():
            pltpu.sync_copy(z_hbm.at[pl.ds(0, _NTAIL)],
                            acc_sh.at[pl.ds(NS * _NR, _NTAIL)])

        pltpu.sync_copy(ei_hbm.at[wid, 0], ei_a)
        pltpu.sync_copy(ei_hbm.at[wid, 1], ei_b)
        pltpu.sync_copy(ei_hbm.at[wid, 2], ei_c)
        plsc.subcore_barrier()

        def _gather(ei, buf, sem):
            pltpu.async_copy(hp_hbm.at[ei.at[0]], buf, sem)

        def _gwait(buf, sem):
            pltpu.make_async_copy(hp_hbm.at[ei_a.at[0]], buf, sem).wait()

        def _scat(ei, buf):
            pltpu.sync_copy(buf, acc_sh.at[ei.at[1]], add=True)

        def _ipre(j, ei, sem):
            pltpu.async_copy(ei_hbm.at[wid, j], ei, sem)

        def _iwait(ei, sem):
            pltpu.make_async_copy(ei_hbm.at[wid, 0], ei, sem).wait()

        _gather(ei_a, rows_a, gsem_a)
        _gather(ei_b, rows_b, gsem_b)

        @pl.loop(0, _NCH - 1, step=3)
        def _(j):
            _gather(ei_c, rows_c, gsem_c)
            _gwait(rows_a, gsem_a)
            _scat(ei_a, rows_a)
            _ipre(j + 3, ei_a, isem_a)
            _gwait(rows_b, gsem_b)
            _scat(ei_b, rows_b)
            _ipre(j + 4, ei_b, isem_b)
            _iwait(ei_a, isem_a)
            _gather(ei_a, rows_a, gsem_a)
            _gwait(rows_c, gsem_c)
            _scat(ei_c, rows_c)
            _ipre(j + 5, ei_c, isem_c)
            _iwait(ei_b, isem_b)
            _gather(ei_b, rows_b, gsem_b)
            _iwait(ei_c, isem_c)

        _gwait(rows_a, gsem_a)
        _scat(ei_a, rows_a)
        _gwait(rows_b, gsem_b)
        plsc.subcore_barrier()
        pltpu.sync_copy(acc_sh.at[pl.ds(r0, _NR)], out_hbm.at[cid, pl.ds(r0, _NR)])

        @pl.when(sid == 0)
        def _():
            pltpu.sync_copy(acc_sh.at[pl.ds(NS * _NR, _NTAIL)],
                            out_hbm.at[cid, pl.ds(NS * _NR, _NTAIL)])

    return k(hp, ei4, zrows)


_BR = 1000


def _mm(a, w):
    return lax.dot_general(a, w, (((1,), (1,)), ((), ())),
                           preferred_element_type=jnp.float32)


def _tc_proj1(x, Wc, Wl, bl):
    def body(x_ref, wc_ref, wl_ref, bl_ref, t_ref, r_ref):
        xb = x_ref[...]
        t_ref[...] = _mm(xb, wc_ref[...])
        r_ref[...] = _mm(xb, wl_ref[...]) + bl_ref[...]

    return pl.pallas_call(
        body,
        grid=(N // _BR,),
        in_specs=[
            pl.BlockSpec((_BR, DIN), lambda i: (i, 0)),
            pl.BlockSpec((DH, DIN), lambda i: (0, 0)),
            pl.BlockSpec((DH, DIN), lambda i: (0, 0)),
            pl.BlockSpec((1, DH), lambda i: (0, 0)),
        ],
        out_specs=[pl.BlockSpec((_BR, DH), lambda i: (i, 0)),
                   pl.BlockSpec((_BR, DH), lambda i: (i, 0))],
        out_shape=[jax.ShapeDtypeStruct((N, DH), jnp.float32),
                   jax.ShapeDtypeStruct((N, DH), jnp.float32)],
    )(x, Wc, Wl, bl)


def _tc_scale(t, hist):
    def body(t_ref, h_ref, hp_ref, dinv_ref):
        deg = h_ref[0][:, 0:1] + h_ref[1][:, 0:1] + 1.0
        dinv = lax.rsqrt(deg)
        dinv_ref[...] = dinv
        hp_ref[...] = t_ref[...] * dinv

    return pl.pallas_call(
        body,
        grid=(N // _BR,),
        in_specs=[
            pl.BlockSpec((_BR, DH), lambda i: (i, 0)),
            pl.BlockSpec((NC, _BR, DH), lambda i: (0, i, 0)),
        ],
        out_specs=[pl.BlockSpec((_BR, DH), lambda i: (i, 0)),
                   pl.BlockSpec((_BR, 1), lambda i: (i, 0))],
        out_shape=[jax.ShapeDtypeStruct((N, DH), jnp.float32),
                   jax.ShapeDtypeStruct((N, 1), jnp.float32)],
    )(t, hist)


def _tc_combine_proj(agg, hp, dinv, r, bc, Wc2, Wl2, bl2):
    din = hp.shape[1]
    dc = Wc2.shape[0]
    dl = Wl2.shape[0]

    def body(agg_ref, hp_ref, dinv_ref, r_ref, bc_ref, wc_ref, wl_ref, bl_ref,
             hp2_ref, r2_ref):
        dinv = dinv_ref[...]
        h = (agg_ref[0] + agg_ref[1] + hp_ref[...]) * dinv + bc_ref[...] + r_ref[...]
        h = jnp.maximum(h, 0.0)
        hp2_ref[...] = _mm(h, wc_ref[...]) * dinv
        r2_ref[...] = _mm(h, wl_ref[...]) + bl_ref[...]

    return pl.pallas_call(
        body,
        grid=(N // _BR,),
        in_specs=[
            pl.BlockSpec((NC, _BR, din), lambda i: (0, i, 0)),
            pl.BlockSpec((_BR, din), lambda i: (i, 0)),
            pl.BlockSpec((_BR, 1), lambda i: (i, 0)),
            pl.BlockSpec((_BR, din), lambda i: (i, 0)),
            pl.BlockSpec((1, din), lambda i: (0, 0)),
            pl.BlockSpec((dc, din), lambda i: (0, 0)),
            pl.BlockSpec((dl, din), lambda i: (0, 0)),
            pl.BlockSpec((1, dl), lambda i: (0, 0)),
        ],
        out_specs=[pl.BlockSpec((_BR, dc), lambda i: (i, 0)),
                   pl.BlockSpec((_BR, dl), lambda i: (i, 0))],
        out_shape=[jax.ShapeDtypeStruct((N, dc), jnp.float32),
                   jax.ShapeDtypeStruct((N, dl), jnp.float32)],
    )(agg, hp, dinv, r, bc, Wc2, Wl2, bl2)


def _tc_final(agg, hp3, dinv, r3, bc3, Wl4, bl4, Wl5, bl5, Wl6, bl6):
    def body(agg_ref, hp_ref, dinv_ref, r_ref, bc_ref, w4_ref, b4_ref,
             w5_ref, b5_ref, w6_ref, b6_ref, out_ref):
        h = ((agg_ref[0][:, :DOUT] + agg_ref[1][:, :DOUT] + hp_ref[:, :DOUT])
             * dinv_ref[...] + bc_ref[...] + r_ref[...])
        m = jnp.max(h, axis=1, keepdims=True)
        e = jnp.exp(h - m)
        z = (h - m) - jnp.log(jnp.sum(e, axis=1, keepdims=True))
        d1 = jnp.maximum(_mm(z, w4_ref[...]) + b4_ref[...], 0.0)
        d2 = jnp.maximum(_mm(d1, w5_ref[...]) + b5_ref[...], 0.0)
        out_ref[...] = _mm(d2, w6_ref[...]) + b6_ref[...]

    return pl.pallas_call(
        body,
        grid=(N // _BR,),
        in_specs=[
            pl.BlockSpec((NC, _BR, DH), lambda i: (0, i, 0)),
            pl.BlockSpec((_BR, DH), lambda i: (i, 0)),
            pl.BlockSpec((_BR, 1), lambda i: (i, 0)),
            pl.BlockSpec((_BR, DOUT), lambda i: (i, 0)),
            pl.BlockSpec((1, DOUT), lambda i: (0, 0)),
            pl.BlockSpec((DH, DOUT), lambda i: (0, 0)),
            pl.BlockSpec((1, DH), lambda i: (0, 0)),
            pl.BlockSpec((DH, DH), lambda i: (0, 0)),
            pl.BlockSpec((1, DH), lambda i: (0, 0)),
            pl.BlockSpec((DIN, DH), lambda i: (0, 0)),
            pl.BlockSpec((1, DIN), lambda i: (0, 0)),
        ],
        out_specs=pl.BlockSpec((_BR, DIN), lambda i: (i, 0)),
        out_shape=jax.ShapeDtypeStruct((N, DIN), jnp.float32),
    )(agg, hp3, dinv, r3, bc3, Wl4, bl4, Wl5, bl5, Wl6, bl6)


def kernel(x, edge_index, Wc1, bc1, Wl1, bl1, Wc2, bc2, Wl2, bl2,
           Wc3, bc3, Wl3, bl3, Wl4, bl4, Wl5, bl5, Wl6, bl6):
    src3 = edge_index[0].reshape(NW, _NCH, _CHUNK)
    dst3 = edge_index[1].reshape(NW, _NCH, _CHUNK)
    ei4 = jnp.stack([src3, dst3], axis=2)
    ei4 = jnp.concatenate(
        [ei4, jnp.zeros((NW, 2, 2, _CHUNK), ei4.dtype)], axis=1)
    ones = jnp.ones((_CHUNK, DH), jnp.float32)
    z128 = jnp.zeros((_NR, DH), jnp.float32)

    hist = _sc_hist(dst3, ones, z128)
    t1, r1 = _tc_proj1(x, Wc1, Wl1, bl1.reshape(1, DH))
    hp1, dinv = _tc_scale(t1, hist)
    agg1 = _sc_conv(hp1, ei4, z128, DH)
    hp2, r2 = _tc_combine_proj(agg1, hp1, dinv, r1, bc1.reshape(1, DH),
                               Wc2, Wl2, bl2.reshape(1, DH))
    agg2 = _sc_conv(hp2, ei4, z128, DH)
    Wc3p = jnp.concatenate([Wc3, jnp.zeros((DH - DOUT, DH), jnp.float32)], axis=0)
    hp3, r3 = _tc_combine_proj(agg2, hp2, dinv, r2, bc2.reshape(1, DH),
                               Wc3p, Wl3, bl3.reshape(1, DOUT))
    agg3 = _sc_conv(hp3, ei4, z128, DH)
    imp = _tc_final(agg3, hp3, dinv, r3, bc3.reshape(1, DOUT),
                    Wl4, bl4.reshape(1, DH), Wl5, bl5.reshape(1, DH),
                    Wl6, bl6.reshape(1, DIN))
    return imp


# speedup vs baseline: 1.4671x; 1.4671x over previous
import functools

import jax
import jax.numpy as jnp
from jax import lax
from jax.experimental import pallas as pl
from jax.experimental.pallas import tpu as pltpu
from jax.experimental.pallas import tpu_sc as plsc

N = 10000
E = 320000
DIN = 128
DH = 128
DOUT = 64

NC = 2
NS = 16
NW = NC * NS
_EPT = E // NW
_CHUNK = 100
_NCH = _EPT // _CHUNK
_NR = 624
_NTAIL = N - NS * _NR


def _mesh():
    return plsc.VectorSubcoreMesh(core_axis_name="c", subcore_axis_name="s")


def _sc_hist(dst3, ones, zrows):
    @functools.partial(
        pl.kernel,
        mesh=_mesh(),
        out_type=jax.ShapeDtypeStruct((NC, N, DH), jnp.float32),
        scratch_types=[
            pltpu.VMEM((_NCH, _CHUNK), jnp.int32),
            pltpu.VMEM((_CHUNK, DH), jnp.float32),
            pltpu.VMEM_SHARED((N, DH), jnp.float32),
        ],
    )
    def k(dst_hbm, ones_hbm, z_hbm, out_hbm, di_v, ones_v, acc_sh):
        cid = lax.axis_index("c")
        sid = lax.axis_index("s")
        wid = cid * NS + sid
        r0 = sid * _NR
        pltpu.sync_copy(z_hbm, acc_sh.at[pl.ds(r0, _NR)])

        @pl.when(sid == 0)
        def _():
            pltpu.sync_copy(z_hbm.at[pl.ds(0, _NTAIL)],
                            acc_sh.at[pl.ds(NS * _NR, _NTAIL)])

        pltpu.sync_copy(ones_hbm, ones_v)
        pltpu.sync_copy(dst_hbm.at[wid], di_v)
        plsc.subcore_barrier()

        @pl.loop(0, _NCH)
        def _(j):
            pltpu.sync_copy(ones_v, acc_sh.at[di_v.at[j]], add=True)

        plsc.subcore_barrier()
        pltpu.sync_copy(acc_sh.at[pl.ds(r0, _NR)], out_hbm.at[cid, pl.ds(r0, _NR)])

        @pl.when(sid == 0)
        def _():
            pltpu.sync_copy(acc_sh.at[pl.ds(NS * _NR, _NTAIL)],
                            out_hbm.at[cid, pl.ds(NS * _NR, _NTAIL)])

    return k(dst3, ones, zrows)


def _sc_conv(hp, ei4, zrows, d):
    @functools.partial(
        pl.kernel,
        mesh=_mesh(),
        out_type=jax.ShapeDtypeStruct((NC, N, d), jnp.float32),
        scratch_types=[
            pltpu.VMEM((2, _CHUNK), jnp.int32),
            pltpu.VMEM((2, _CHUNK), jnp.int32),
            pltpu.VMEM((_CHUNK, d), jnp.float32),
            pltpu.VMEM((_CHUNK, d), jnp.float32),
            pltpu.VMEM_SHARED((N, d), jnp.float32),
            pltpu.SemaphoreType.DMA,
            pltpu.SemaphoreType.DMA,
            pltpu.SemaphoreType.DMA,
            pltpu.SemaphoreType.DMA,
        ],
    )
    def k(hp_hbm, ei_hbm, z_hbm, out_hbm, ei_a, ei_b,
          rows_a, rows_b, acc_sh, gsem_a, gsem_b, isem_a, isem_b):
        cid = lax.axis_index("c")
        sid = lax.axis_index("s")
        wid = cid * NS + sid
        r0 = sid * _NR
        pltpu.sync_copy(z_hbm, acc_sh.at[pl.ds(r0, _NR)])

        @pl.when(sid == 0)
        def _():
            pltpu.sync_copy(z_hbm.at[pl.ds(0, _NTAIL)],
                            acc_sh.at[pl.ds(NS * _NR, _NTAIL)])

        pltpu.sync_copy(ei_hbm.at[wid, 0], ei_a)
        pltpu.sync_copy(ei_hbm.at[wid, 1], ei_b)
        plsc.subcore_barrier()

        def _gather(ei, buf, sem):
            pltpu.async_copy(hp_hbm.at[ei.at[0]], buf, sem)

        def _gwait(buf, sem):
            pltpu.make_async_copy(hp_hbm.at[ei_a.at[0]], buf, sem).wait()

        def _scat(ei, buf):
            pltpu.sync_copy(buf, acc_sh.at[ei.at[1]], add=True)

        _gather(ei_a, rows_a, gsem_a)

        @pl.loop(0, _NCH - 2, step=2)
        def _(j):
            _gather(ei_b, rows_b, gsem_b)
            _gwait(rows_a, gsem_a)
            _scat(ei_a, rows_a)
            pltpu.async_copy(ei_hbm.at[wid, j + 2], ei_a, isem_a)
            pltpu.make_async_copy(ei_hbm.at[wid, 0], ei_a, isem_a).wait()
            _gwait(rows_b, gsem_b)
            _gather(ei_a, rows_a, gsem_a)
            _scat(ei_b, rows_b)
            pltpu.async_copy(ei_hbm.at[wid, j + 3], ei_b, isem_b)
            pltpu.make_async_copy(ei_hbm.at[wid, 0], ei_b, isem_b).wait()

        _gather(ei_b, rows_b, gsem_b)
        _gwait(rows_a, gsem_a)
        _scat(ei_a, rows_a)
        _gwait(rows_b, gsem_b)
        _scat(ei_b, rows_b)
        plsc.subcore_barrier()
        pltpu.sync_copy(acc_sh.at[pl.ds(r0, _NR)], out_hbm.at[cid, pl.ds(r0, _NR)])

        @pl.when(sid == 0)
        def _():
            pltpu.sync_copy(acc_sh.at[pl.ds(NS * _NR, _NTAIL)],
                            out_hbm.at[cid, pl.ds(NS * _NR, _NTAIL)])

    return k(hp, ei4, zrows)


_BR = 1000


def _mm(a, w):
    return lax.dot_general(a, w, (((1,), (1,)), ((), ())),
                           preferred_element_type=jnp.float32)


def _tc_proj1(x, Wc, Wl, bl):
    def body(x_ref, wc_ref, wl_ref, bl_ref, t_ref, r_ref):
        xb = x_ref[...]
        t_ref[...] = _mm(xb, wc_ref[...])
        r_ref[...] = _mm(xb, wl_ref[...]) + bl_ref[...]

    return pl.pallas_call(
        body,
        grid=(N // _BR,),
        in_specs=[
            pl.BlockSpec((_BR, DIN), lambda i: (i, 0)),
            pl.BlockSpec((DH, DIN), lambda i: (0, 0)),
            pl.BlockSpec((DH, DIN), lambda i: (0, 0)),
            pl.BlockSpec((1, DH), lambda i: (0, 0)),
        ],
        out_specs=[pl.BlockSpec((_BR, DH), lambda i: (i, 0)),
                   pl.BlockSpec((_BR, DH), lambda i: (i, 0))],
        out_shape=[jax.ShapeDtypeStruct((N, DH), jnp.float32),
                   jax.ShapeDtypeStruct((N, DH), jnp.float32)],
    )(x, Wc, Wl, bl)


def _tc_scale(t, hist):
    def body(t_ref, h_ref, hp_ref, dinv_ref):
        deg = h_ref[0][:, 0:1] + h_ref[1][:, 0:1] + 1.0
        dinv = lax.rsqrt(deg)
        dinv_ref[...] = dinv
        hp_ref[...] = t_ref[...] * dinv

    return pl.pallas_call(
        body,
        grid=(N // _BR,),
        in_specs=[
            pl.BlockSpec((_BR, DH), lambda i: (i, 0)),
            pl.BlockSpec((NC, _BR, DH), lambda i: (0, i, 0)),
        ],
        out_specs=[pl.BlockSpec((_BR, DH), lambda i: (i, 0)),
                   pl.BlockSpec((_BR, 1), lambda i: (i, 0))],
        out_shape=[jax.ShapeDtypeStruct((N, DH), jnp.float32),
                   jax.ShapeDtypeStruct((N, 1), jnp.float32)],
    )(t, hist)


def _tc_combine_proj(agg, hp, dinv, r, bc, Wc2, Wl2, bl2):
    din = hp.shape[1]
    dc = Wc2.shape[0]
    dl = Wl2.shape[0]

    def body(agg_ref, hp_ref, dinv_ref, r_ref, bc_ref, wc_ref, wl_ref, bl_ref,
             hp2_ref, r2_ref):
        dinv = dinv_ref[...]
        h = (agg_ref[0] + agg_ref[1] + hp_ref[...]) * dinv + bc_ref[...] + r_ref[...]
        h = jnp.maximum(h, 0.0)
        hp2_ref[...] = _mm(h, wc_ref[...]) * dinv
        r2_ref[...] = _mm(h, wl_ref[...]) + bl_ref[...]

    return pl.pallas_call(
        body,
        grid=(N // _BR,),
        in_specs=[
            pl.BlockSpec((NC, _BR, din), lambda i: (0, i, 0)),
            pl.BlockSpec((_BR, din), lambda i: (i, 0)),
            pl.BlockSpec((_BR, 1), lambda i: (i, 0)),
            pl.BlockSpec((_BR, din), lambda i: (i, 0)),
            pl.BlockSpec((1, din), lambda i: (0, 0)),
            pl.BlockSpec((dc, din), lambda i: (0, 0)),
            pl.BlockSpec((dl, din), lambda i: (0, 0)),
            pl.BlockSpec((1, dl), lambda i: (0, 0)),
        ],
        out_specs=[pl.BlockSpec((_BR, dc), lambda i: (i, 0)),
                   pl.BlockSpec((_BR, dl), lambda i: (i, 0))],
        out_shape=[jax.ShapeDtypeStruct((N, dc), jnp.float32),
                   jax.ShapeDtypeStruct((N, dl), jnp.float32)],
    )(agg, hp, dinv, r, bc, Wc2, Wl2, bl2)


def _tc_final(agg, hp3, dinv, r3, bc3, Wl4, bl4, Wl5, bl5, Wl6, bl6):
    def body(agg_ref, hp_ref, dinv_ref, r_ref, bc_ref, w4_ref, b4_ref,
             w5_ref, b5_ref, w6_ref, b6_ref, out_ref):
        h = ((agg_ref[0][:, :DOUT] + agg_ref[1][:, :DOUT] + hp_ref[:, :DOUT])
             * dinv_ref[...] + bc_ref[...] + r_ref[...])
        m = jnp.max(h, axis=1, keepdims=True)
        e = jnp.exp(h - m)
        z = (h - m) - jnp.log(jnp.sum(e, axis=1, keepdims=True))
        d1 = jnp.maximum(_mm(z, w4_ref[...]) + b4_ref[...], 0.0)
        d2 = jnp.maximum(_mm(d1, w5_ref[...]) + b5_ref[...], 0.0)
        out_ref[...] = _mm(d2, w6_ref[...]) + b6_ref[...]

    return pl.pallas_call(
        body,
        grid=(N // _BR,),
        in_specs=[
            pl.BlockSpec((NC, _BR, DH), lambda i: (0, i, 0)),
            pl.BlockSpec((_BR, DH), lambda i: (i, 0)),
            pl.BlockSpec((_BR, 1), lambda i: (i, 0)),
            pl.BlockSpec((_BR, DOUT), lambda i: (i, 0)),
            pl.BlockSpec((1, DOUT), lambda i: (0, 0)),
            pl.BlockSpec((DH, DOUT), lambda i: (0, 0)),
            pl.BlockSpec((1, DH), lambda i: (0, 0)),
            pl.BlockSpec((DH, DH), lambda i: (0, 0)),
            pl.BlockSpec((1, DH), lambda i: (0, 0)),
            pl.BlockSpec((DIN, DH), lambda i: (0, 0)),
            pl.BlockSpec((1, DIN), lambda i: (0, 0)),
        ],
        out_specs=pl.BlockSpec((_BR, DIN), lambda i: (i, 0)),
        out_shape=jax.ShapeDtypeStruct((N, DIN), jnp.float32),
    )(agg, hp3, dinv, r3, bc3, Wl4, bl4, Wl5, bl5, Wl6, bl6)


def kernel(x, edge_index, Wc1, bc1, Wl1, bl1, Wc2, bc2, Wl2, bl2,
           Wc3, bc3, Wl3, bl3, Wl4, bl4, Wl5, bl5, Wl6, bl6):
    src3 = edge_index[0].reshape(NW, _NCH, _CHUNK)
    dst3 = edge_index[1].reshape(NW, _NCH, _CHUNK)
    ei4 = jnp.stack([src3, dst3], axis=2)
    ones = jnp.ones((_CHUNK, DH), jnp.float32)
    z128 = jnp.zeros((_NR, DH), jnp.float32)

    hist = _sc_hist(dst3, ones, z128)
    t1, r1 = _tc_proj1(x, Wc1, Wl1, bl1.reshape(1, DH))
    hp1, dinv = _tc_scale(t1, hist)
    agg1 = _sc_conv(hp1, ei4, z128, DH)
    hp2, r2 = _tc_combine_proj(agg1, hp1, dinv, r1, bc1.reshape(1, DH),
                               Wc2, Wl2, bl2.reshape(1, DH))
    agg2 = _sc_conv(hp2, ei4, z128, DH)
    Wc3p = jnp.concatenate([Wc3, jnp.zeros((DH - DOUT, DH), jnp.float32)], axis=0)
    hp3, r3 = _tc_combine_proj(agg2, hp2, dinv, r2, bc2.reshape(1, DH),
                               Wc3p, Wl3, bl3.reshape(1, DOUT))
    agg3 = _sc_conv(hp3, ei4, z128, DH)
    imp = _tc_final(agg3, hp3, dinv, r3, bc3.reshape(1, DOUT),
                    Wl4, bl4.reshape(1, DH), Wl5, bl5.reshape(1, DH),
                    Wl6, bl6.reshape(1, DIN))
    return imp


# speedup vs baseline: 1.4674x; 1.0002x over previous
import functools

import jax
import jax.numpy as jnp
from jax import lax
from jax.experimental import pallas as pl
from jax.experimental.pallas import tpu as pltpu
from jax.experimental.pallas import tpu_sc as plsc

N = 10000
E = 320000
DIN = 128
DH = 128
DOUT = 64

NC = 2
NS = 16
NW = NC * NS
_EPT = E // NW
_CHUNK = 100
_NCH = _EPT // _CHUNK
_NR = 624
_NTAIL = N - NS * _NR


def _mesh():
    return plsc.VectorSubcoreMesh(core_axis_name="c", subcore_axis_name="s")


def _sc_hist(dst3, ones, zrows):
    @functools.partial(
        pl.kernel,
        mesh=_mesh(),
        out_type=jax.ShapeDtypeStruct((NC, N, DH), jnp.float32),
        scratch_types=[
            pltpu.VMEM((_NCH, _CHUNK), jnp.int32),
            pltpu.VMEM((_CHUNK, DH), jnp.float32),
            pltpu.VMEM_SHARED((N, DH), jnp.float32),
        ],
    )
    def k(dst_hbm, ones_hbm, z_hbm, out_hbm, di_v, ones_v, acc_sh):
        cid = lax.axis_index("c")
        sid = lax.axis_index("s")
        wid = cid * NS + sid
        r0 = sid * _NR
        pltpu.sync_copy(z_hbm, acc_sh.at[pl.ds(r0, _NR)])

        @pl.when(sid == 0)
        def _():
            pltpu.sync_copy(z_hbm.at[pl.ds(0, _NTAIL)],
                            acc_sh.at[pl.ds(NS * _NR, _NTAIL)])

        pltpu.sync_copy(ones_hbm, ones_v)
        pltpu.sync_copy(dst_hbm.at[wid], di_v)
        plsc.subcore_barrier()

        @pl.loop(0, _NCH)
        def _(j):
            pltpu.sync_copy(ones_v, acc_sh.at[di_v.at[j]], add=True)

        plsc.subcore_barrier()
        pltpu.sync_copy(acc_sh.at[pl.ds(r0, _NR)], out_hbm.at[cid, pl.ds(r0, _NR)])

        @pl.when(sid == 0)
        def _():
            pltpu.sync_copy(acc_sh.at[pl.ds(NS * _NR, _NTAIL)],
                            out_hbm.at[cid, pl.ds(NS * _NR, _NTAIL)])

    return k(dst3, ones, zrows)


def _sc_conv(hp, ei4, zrows, d):
    @functools.partial(
        pl.kernel,
        mesh=_mesh(),
        out_type=jax.ShapeDtypeStruct((NC, N, d), jnp.float32),
        scratch_types=[
            pltpu.VMEM((2, _CHUNK), jnp.int32),
            pltpu.VMEM((2, _CHUNK), jnp.int32),
            pltpu.VMEM((_CHUNK, d), jnp.float32),
            pltpu.VMEM((_CHUNK, d), jnp.float32),
            pltpu.VMEM_SHARED((N, d), jnp.float32),
            pltpu.SemaphoreType.DMA,
            pltpu.SemaphoreType.DMA,
            pltpu.SemaphoreType.DMA,
            pltpu.SemaphoreType.DMA,
        ],
    )
    def k(hp_hbm, ei_hbm, z_hbm, out_hbm, ei_a, ei_b,
          rows_a, rows_b, acc_sh, gsem_a, gsem_b, isem_a, isem_b):
        cid = lax.axis_index("c")
        sid = lax.axis_index("s")
        wid = cid * NS + sid
        r0 = sid * _NR
        pltpu.sync_copy(z_hbm, acc_sh.at[pl.ds(r0, _NR)])

        @pl.when(sid == 0)
        def _():
            pltpu.sync_copy(z_hbm.at[pl.ds(0, _NTAIL)],
                            acc_sh.at[pl.ds(NS * _NR, _NTAIL)])

        pltpu.sync_copy(ei_hbm.at[wid, 0], ei_a)
        pltpu.sync_copy(ei_hbm.at[wid, 1], ei_b)
        plsc.subcore_barrier()

        def _gather(ei, buf, sem):
            pltpu.async_copy(hp_hbm.at[ei.at[0]], buf, sem)

        def _gwait(buf, sem):
            pltpu.make_async_copy(hp_hbm.at[ei_a.at[0]], buf, sem).wait()

        def _scat(ei, buf):
            pltpu.sync_copy(buf, acc_sh.at[ei.at[1]], add=True)

        _gather(ei_a, rows_a, gsem_a)

        @pl.loop(0, _NCH - 2, step=2)
        def _(j):
            _gather(ei_b, rows_b, gsem_b)
            _gwait(rows_a, gsem_a)
            _scat(ei_a, rows_a)
            pltpu.async_copy(ei_hbm.at[wid, j + 2], ei_a, isem_a)
            pltpu.make_async_copy(ei_hbm.at[wid, 0], ei_a, isem_a).wait()
            _gwait(rows_b, gsem_b)
            _gather(ei_a, rows_a, gsem_a)
            _scat(ei_b, rows_b)
            pltpu.async_copy(ei_hbm.at[wid, j + 3], ei_b, isem_b)
            pltpu.make_async_copy(ei_hbm.at[wid, 0], ei_b, isem_b).wait()

        _gather(ei_b, rows_b, gsem_b)
        _gwait(rows_a, gsem_a)
        _scat(ei_a, rows_a)
        _gwait(rows_b, gsem_b)
        _scat(ei_b, rows_b)
        plsc.subcore_barrier()
        pltpu.sync_copy(acc_sh.at[pl.ds(r0, _NR)], out_hbm.at[cid, pl.ds(r0, _NR)])

        @pl.when(sid == 0)
        def _():
            pltpu.sync_copy(acc_sh.at[pl.ds(NS * _NR, _NTAIL)],
                            out_hbm.at[cid, pl.ds(NS * _NR, _NTAIL)])

    return k(hp, ei4, zrows)


_BR = 1000


def _mm(a, w):
    return lax.dot_general(a, w, (((1,), (1,)), ((), ())),
                           preferred_element_type=jnp.float32)


def _tc_proj1(x, Wc, Wl, bl):
    def body(x_ref, wc_ref, wl_ref, bl_ref, t_ref, r_ref):
        xb = x_ref[...]
        t_ref[...] = _mm(xb, wc_ref[...])
        r_ref[...] = _mm(xb, wl_ref[...]) + bl_ref[...]

    return pl.pallas_call(
        body,
        grid=(N // _BR,),
        in_specs=[
            pl.BlockSpec((_BR, DIN), lambda i: (i, 0)),
            pl.BlockSpec((DH, DIN), lambda i: (0, 0)),
            pl.BlockSpec((DH, DIN), lambda i: (0, 0)),
            pl.BlockSpec((1, DH), lambda i: (0, 0)),
        ],
        out_specs=[pl.BlockSpec((_BR, DH), lambda i: (i, 0)),
                   pl.BlockSpec((_BR, DH), lambda i: (i, 0))],
        out_shape=[jax.ShapeDtypeStruct((N, DH), jnp.float32),
                   jax.ShapeDtypeStruct((N, DH), jnp.float32)],
    )(x, Wc, Wl, bl)


def _tc_scale(t, hist):
    def body(t_ref, h_ref, hp_ref, dinv_ref):
        deg = h_ref[0][:, 0:1] + h_ref[1][:, 0:1] + 1.0
        dinv = lax.rsqrt(deg)
        dinv_ref[...] = dinv
        hp_ref[...] = t_ref[...] * dinv

    return pl.pallas_call(
        body,
        grid=(N // _BR,),
        in_specs=[
            pl.BlockSpec((_BR, DH), lambda i: (i, 0)),
            pl.BlockSpec((NC, _BR, DH), lambda i: (0, i, 0)),
        ],
        out_specs=[pl.BlockSpec((_BR, DH), lambda i: (i, 0)),
                   pl.BlockSpec((_BR, 1), lambda i: (i, 0))],
        out_shape=[jax.ShapeDtypeStruct((N, DH), jnp.float32),
                   jax.ShapeDtypeStruct((N, 1), jnp.float32)],
    )(t, hist)


def _tc_combine_proj(agg, hp, dinv, r, bc, Wc2, Wl2, bl2):
    din = hp.shape[1]
    dc = Wc2.shape[0]
    dl = Wl2.shape[0]

    def body(agg_ref, hp_ref, dinv_ref, r_ref, bc_ref, wc_ref, wl_ref, bl_ref,
             hp2_ref, r2_ref):
        dinv = dinv_ref[...]
        h = (agg_ref[0] + agg_ref[1] + hp_ref[...]) * dinv + bc_ref[...] + r_ref[...]
        h = jnp.maximum(h, 0.0)
        hp2_ref[...] = _mm(h, wc_ref[...]) * dinv
        r2_ref[...] = _mm(h, wl_ref[...]) + bl_ref[...]

    return pl.pallas_call(
        body,
        grid=(N // _BR,),
        in_specs=[
            pl.BlockSpec((NC, _BR, din), lambda i: (0, i, 0)),
            pl.BlockSpec((_BR, din), lambda i: (i, 0)),
            pl.BlockSpec((_BR, 1), lambda i: (i, 0)),
            pl.BlockSpec((_BR, din), lambda i: (i, 0)),
            pl.BlockSpec((1, din), lambda i: (0, 0)),
            pl.BlockSpec((dc, din), lambda i: (0, 0)),
            pl.BlockSpec((dl, din), lambda i: (0, 0)),
            pl.BlockSpec((1, dl), lambda i: (0, 0)),
        ],
        out_specs=[pl.BlockSpec((_BR, dc), lambda i: (i, 0)),
                   pl.BlockSpec((_BR, dl), lambda i: (i, 0))],
        out_shape=[jax.ShapeDtypeStruct((N, dc), jnp.float32),
                   jax.ShapeDtypeStruct((N, dl), jnp.float32)],
    )(agg, hp, dinv, r, bc, Wc2, Wl2, bl2)


def _tc_final(agg, hp3, dinv, r3, bc3, Wl4, bl4, Wl5, bl5, Wl6, bl6):
    def body(agg_ref, hp_ref, dinv_ref, r_ref, bc_ref, w4_ref, b4_ref,
             w5_ref, b5_ref, w6_ref, b6_ref, out_ref):
        h = ((agg_ref[0][:, :DOUT] + agg_ref[1][:, :DOUT] + hp_ref[:, :DOUT])
             * dinv_ref[...] + bc_ref[...] + r_ref[...])
        m = jnp.max(h, axis=1, keepdims=True)
        e = jnp.exp(h - m)
        z = (h - m) - jnp.log(jnp.sum(e, axis=1, keepdims=True))
        d1 = jnp.maximum(_mm(z, w4_ref[...]) + b4_ref[...], 0.0)
        d2 = jnp.maximum(_mm(d1, w5_ref[...]) + b5_ref[...], 0.0)
        out_ref[...] = _mm(d2, w6_ref[...]) + b6_ref[...]

    return pl.pallas_call(
        body,
        grid=(N // _BR,),
        in_specs=[
            pl.BlockSpec((NC, _BR, DH), lambda i: (0, i, 0)),
            pl.BlockSpec((_BR, DH), lambda i: (i, 0)),
            pl.BlockSpec((_BR, 1), lambda i: (i, 0)),
            pl.BlockSpec((_BR, DOUT), lambda i: (i, 0)),
            pl.BlockSpec((1, DOUT), lambda i: (0, 0)),
            pl.BlockSpec((DH, DOUT), lambda i: (0, 0)),
            pl.BlockSpec((1, DH), lambda i: (0, 0)),
            pl.BlockSpec((DH, DH), lambda i: (0, 0)),
            pl.BlockSpec((1, DH), lambda i: (0, 0)),
            pl.BlockSpec((DIN, DH), lambda i: (0, 0)),
            pl.BlockSpec((1, DIN), lambda i: (0, 0)),
        ],
        out_specs=pl.BlockSpec((_BR, DIN), lambda i: (i, 0)),
        out_shape=jax.ShapeDtypeStruct((N, DIN), jnp.float32),
    )(agg, hp3, dinv, r3, bc3, Wl4, bl4, Wl5, bl5, Wl6, bl6)


def kernel(x, edge_index, Wc1, bc1, Wl1, bl1, Wc2, bc2, Wl2, bl2,
           Wc3, bc3, Wl3, bl3, Wl4, bl4, Wl5, bl5, Wl6, bl6):
    edge_index = edge_index.astype(jnp.int32)
    src3 = edge_index[0].reshape(NW, _NCH, _CHUNK)
    dst3 = edge_index[1].reshape(NW, _NCH, _CHUNK)
    ei4 = jnp.stack([src3, dst3], axis=2)
    ones = jnp.ones((_CHUNK, DH), jnp.float32)
    z128 = jnp.zeros((_NR, DH), jnp.float32)

    hist = _sc_hist(dst3, ones, z128)
    t1, r1 = _tc_proj1(x, Wc1, Wl1, bl1.reshape(1, DH))
    hp1, dinv = _tc_scale(t1, hist)
    agg1 = _sc_conv(hp1, ei4, z128, DH)
    hp2, r2 = _tc_combine_proj(agg1, hp1, dinv, r1, bc1.reshape(1, DH),
                               Wc2, Wl2, bl2.reshape(1, DH))
    agg2 = _sc_conv(hp2, ei4, z128, DH)
    Wc3p = jnp.concatenate([Wc3, jnp.zeros((DH - DOUT, DH), jnp.float32)], axis=0)
    hp3, r3 = _tc_combine_proj(agg2, hp2, dinv, r2, bc2.reshape(1, DH),
                               Wc3p, Wl3, bl3.reshape(1, DOUT))
    agg3 = _sc_conv(hp3, ei4, z128, DH)
    imp = _tc_final(agg3, hp3, dinv, r3, bc3.reshape(1, DOUT),
                    Wl4, bl4.reshape(1, DH), Wl5, bl5.reshape(1, DH),
                    Wl6, bl6.reshape(1, DIN))
    return imp


# speedup vs baseline: 1.4704x; 1.0021x over previous
import functools

import jax
import jax.numpy as jnp
from jax import lax
from jax.experimental import pallas as pl
from jax.experimental.pallas import tpu as pltpu
from jax.experimental.pallas import tpu_sc as plsc

N = 10000
E = 320000
DIN = 128
DH = 128
DOUT = 64

NC = 2
NS = 16
NW = NC * NS
_EPT = E // NW
_CHUNK = 100
_NCH = _EPT // _CHUNK
_NR = 624
_NTAIL = N - NS * _NR


def _mesh():
    return plsc.VectorSubcoreMesh(core_axis_name="c", subcore_axis_name="s")


def _sc_hist(dst3, ones, zrows):
    @functools.partial(
        pl.kernel,
        mesh=_mesh(),
        out_type=jax.ShapeDtypeStruct((NC, N, DH), jnp.float32),
        scratch_types=[
            pltpu.VMEM((_NCH, _CHUNK), jnp.int32),
            pltpu.VMEM((_CHUNK, DH), jnp.float32),
            pltpu.VMEM_SHARED((N, DH), jnp.float32),
            pltpu.SemaphoreType.DMA,
        ],
    )
    def k(dst_hbm, ones_hbm, z_hbm, out_hbm, di_v, ones_v, acc_sh, sem):
        cid = lax.axis_index("c")
        sid = lax.axis_index("s")
        wid = cid * NS + sid
        r0 = sid * _NR
        pltpu.sync_copy(z_hbm, acc_sh.at[pl.ds(r0, _NR)])

        @pl.when(sid == 0)
        def _():
            pltpu.sync_copy(z_hbm.at[pl.ds(0, _NTAIL)],
                            acc_sh.at[pl.ds(NS * _NR, _NTAIL)])

        pltpu.sync_copy(ones_hbm, ones_v)
        pltpu.sync_copy(dst_hbm.at[wid], di_v)
        plsc.subcore_barrier()

        pltpu.async_copy(ones_v, acc_sh.at[di_v.at[0]], sem, add=True)

        @pl.loop(1, _NCH)
        def _(j):
            pltpu.async_copy(ones_v, acc_sh.at[di_v.at[j]], sem, add=True)
            pltpu.make_async_copy(ones_v, acc_sh.at[di_v.at[0]], sem).wait()

        pltpu.make_async_copy(ones_v, acc_sh.at[di_v.at[0]], sem).wait()
        plsc.subcore_barrier()
        pltpu.sync_copy(acc_sh.at[pl.ds(r0, _NR)], out_hbm.at[cid, pl.ds(r0, _NR)])

        @pl.when(sid == 0)
        def _():
            pltpu.sync_copy(acc_sh.at[pl.ds(NS * _NR, _NTAIL)],
                            out_hbm.at[cid, pl.ds(NS * _NR, _NTAIL)])

    return k(dst3, ones, zrows)


def _sc_conv(hp, ei4, zrows, d):
    @functools.partial(
        pl.kernel,
        mesh=_mesh(),
        out_type=jax.ShapeDtypeStruct((NC, N, d), jnp.float32),
        scratch_types=[
            pltpu.VMEM((2, _CHUNK), jnp.int32),
            pltpu.VMEM((2, _CHUNK), jnp.int32),
            pltpu.VMEM((_CHUNK, d), jnp.float32),
            pltpu.VMEM((_CHUNK, d), jnp.float32),
            pltpu.VMEM_SHARED((N, d), jnp.float32),
            pltpu.SemaphoreType.DMA,
            pltpu.SemaphoreType.DMA,
            pltpu.SemaphoreType.DMA,
            pltpu.SemaphoreType.DMA,
        ],
    )
    def k(hp_hbm, ei_hbm, z_hbm, out_hbm, ei_a, ei_b,
          rows_a, rows_b, acc_sh, gsem_a, gsem_b, isem_a, isem_b):
        cid = lax.axis_index("c")
        sid = lax.axis_index("s")
        wid = cid * NS + sid
        r0 = sid * _NR
        pltpu.sync_copy(z_hbm, acc_sh.at[pl.ds(r0, _NR)])

        @pl.when(sid == 0)
        def _():
            pltpu.sync_copy(z_hbm.at[pl.ds(0, _NTAIL)],
                            acc_sh.at[pl.ds(NS * _NR, _NTAIL)])

        pltpu.sync_copy(ei_hbm.at[wid, 0], ei_a)
        pltpu.sync_copy(ei_hbm.at[wid, 1], ei_b)
        plsc.subcore_barrier()

        def _gather(ei, buf, sem):
            pltpu.async_copy(hp_hbm.at[ei.at[0]], buf, sem)

        def _gwait(buf, sem):
            pltpu.make_async_copy(hp_hbm.at[ei_a.at[0]], buf, sem).wait()

        def _scat(ei, buf):
            pltpu.sync_copy(buf, acc_sh.at[ei.at[1]], add=True)

        _gather(ei_a, rows_a, gsem_a)

        @pl.loop(0, _NCH - 2, step=2)
        def _(j):
            _gather(ei_b, rows_b, gsem_b)
            _gwait(rows_a, gsem_a)
            _scat(ei_a, rows_a)
            pltpu.async_copy(ei_hbm.at[wid, j + 2], ei_a, isem_a)
            pltpu.make_async_copy(ei_hbm.at[wid, 0], ei_a, isem_a).wait()
            _gwait(rows_b, gsem_b)
            _gather(ei_a, rows_a, gsem_a)
            _scat(ei_b, rows_b)
            pltpu.async_copy(ei_hbm.at[wid, j + 3], ei_b, isem_b)
            pltpu.make_async_copy(ei_hbm.at[wid, 0], ei_b, isem_b).wait()

        _gather(ei_b, rows_b, gsem_b)
        _gwait(rows_a, gsem_a)
        _scat(ei_a, rows_a)
        _gwait(rows_b, gsem_b)
        _scat(ei_b, rows_b)
        plsc.subcore_barrier()
        pltpu.sync_copy(acc_sh.at[pl.ds(r0, _NR)], out_hbm.at[cid, pl.ds(r0, _NR)])

        @pl.when(sid == 0)
        def _():
            pltpu.sync_copy(acc_sh.at[pl.ds(NS * _NR, _NTAIL)],
                            out_hbm.at[cid, pl.ds(NS * _NR, _NTAIL)])

    return k(hp, ei4, zrows)


_BR = 1000


def _mm(a, w):
    return lax.dot_general(a, w, (((1,), (1,)), ((), ())),
                           preferred_element_type=jnp.float32)


def _tc_proj1(x, Wc, Wl, bl):
    def body(x_ref, wc_ref, wl_ref, bl_ref, t_ref, r_ref):
        xb = x_ref[...]
        t_ref[...] = _mm(xb, wc_ref[...])
        r_ref[...] = _mm(xb, wl_ref[...]) + bl_ref[...]

    return pl.pallas_call(
        body,
        grid=(N // _BR,),
        in_specs=[
            pl.BlockSpec((_BR, DIN), lambda i: (i, 0)),
            pl.BlockSpec((DH, DIN), lambda i: (0, 0)),
            pl.BlockSpec((DH, DIN), lambda i: (0, 0)),
            pl.BlockSpec((1, DH), lambda i: (0, 0)),
        ],
        out_specs=[pl.BlockSpec((_BR, DH), lambda i: (i, 0)),
                   pl.BlockSpec((_BR, DH), lambda i: (i, 0))],
        out_shape=[jax.ShapeDtypeStruct((N, DH), jnp.float32),
                   jax.ShapeDtypeStruct((N, DH), jnp.float32)],
    )(x, Wc, Wl, bl)


def _tc_scale(t, hist):
    def body(t_ref, h_ref, hp_ref, dinv_ref):
        deg = h_ref[0][:, 0:1] + h_ref[1][:, 0:1] + 1.0
        dinv = lax.rsqrt(deg)
        dinv_ref[...] = dinv
        hp_ref[...] = t_ref[...] * dinv

    return pl.pallas_call(
        body,
        grid=(N // _BR,),
        in_specs=[
            pl.BlockSpec((_BR, DH), lambda i: (i, 0)),
            pl.BlockSpec((NC, _BR, DH), lambda i: (0, i, 0)),
        ],
        out_specs=[pl.BlockSpec((_BR, DH), lambda i: (i, 0)),
                   pl.BlockSpec((_BR, 1), lambda i: (i, 0))],
        out_shape=[jax.ShapeDtypeStruct((N, DH), jnp.float32),
                   jax.ShapeDtypeStruct((N, 1), jnp.float32)],
    )(t, hist)


def _tc_combine_proj(agg, hp, dinv, r, bc, Wc2, Wl2, bl2):
    din = hp.shape[1]
    dc = Wc2.shape[0]
    dl = Wl2.shape[0]

    def body(agg_ref, hp_ref, dinv_ref, r_ref, bc_ref, wc_ref, wl_ref, bl_ref,
             hp2_ref, r2_ref):
        dinv = dinv_ref[...]
        h = (agg_ref[0] + agg_ref[1] + hp_ref[...]) * dinv + bc_ref[...] + r_ref[...]
        h = jnp.maximum(h, 0.0)
        hp2_ref[...] = _mm(h, wc_ref[...]) * dinv
        r2_ref[...] = _mm(h, wl_ref[...]) + bl_ref[...]

    return pl.pallas_call(
        body,
        grid=(N // _BR,),
        in_specs=[
            pl.BlockSpec((NC, _BR, din), lambda i: (0, i, 0)),
            pl.BlockSpec((_BR, din), lambda i: (i, 0)),
            pl.BlockSpec((_BR, 1), lambda i: (i, 0)),
            pl.BlockSpec((_BR, din), lambda i: (i, 0)),
            pl.BlockSpec((1, din), lambda i: (0, 0)),
            pl.BlockSpec((dc, din), lambda i: (0, 0)),
            pl.BlockSpec((dl, din), lambda i: (0, 0)),
            pl.BlockSpec((1, dl), lambda i: (0, 0)),
        ],
        out_specs=[pl.BlockSpec((_BR, dc), lambda i: (i, 0)),
                   pl.BlockSpec((_BR, dl), lambda i: (i, 0))],
        out_shape=[jax.ShapeDtypeStruct((N, dc), jnp.float32),
                   jax.ShapeDtypeStruct((N, dl), jnp.float32)],
    )(agg, hp, dinv, r, bc, Wc2, Wl2, bl2)


def _tc_final(agg, hp3, dinv, r3, bc3, Wl4, bl4, Wl5, bl5, Wl6, bl6):
    def body(agg_ref, hp_ref, dinv_ref, r_ref, bc_ref, w4_ref, b4_ref,
             w5_ref, b5_ref, w6_ref, b6_ref, out_ref):
        h = ((agg_ref[0][:, :DOUT] + agg_ref[1][:, :DOUT] + hp_ref[:, :DOUT])
             * dinv_ref[...] + bc_ref[...] + r_ref[...])
        m = jnp.max(h, axis=1, keepdims=True)
        e = jnp.exp(h - m)
        z = (h - m) - jnp.log(jnp.sum(e, axis=1, keepdims=True))
        d1 = jnp.maximum(_mm(z, w4_ref[...]) + b4_ref[...], 0.0)
        d2 = jnp.maximum(_mm(d1, w5_ref[...]) + b5_ref[...], 0.0)
        out_ref[...] = _mm(d2, w6_ref[...]) + b6_ref[...]

    return pl.pallas_call(
        body,
        grid=(N // _BR,),
        in_specs=[
            pl.BlockSpec((NC, _BR, DH), lambda i: (0, i, 0)),
            pl.BlockSpec((_BR, DH), lambda i: (i, 0)),
            pl.BlockSpec((_BR, 1), lambda i: (i, 0)),
            pl.BlockSpec((_BR, DOUT), lambda i: (i, 0)),
            pl.BlockSpec((1, DOUT), lambda i: (0, 0)),
            pl.BlockSpec((DH, DOUT), lambda i: (0, 0)),
            pl.BlockSpec((1, DH), lambda i: (0, 0)),
            pl.BlockSpec((DH, DH), lambda i: (0, 0)),
            pl.BlockSpec((1, DH), lambda i: (0, 0)),
            pl.BlockSpec((DIN, DH), lambda i: (0, 0)),
            pl.BlockSpec((1, DIN), lambda i: (0, 0)),
        ],
        out_specs=pl.BlockSpec((_BR, DIN), lambda i: (i, 0)),
        out_shape=jax.ShapeDtypeStruct((N, DIN), jnp.float32),
    )(agg, hp3, dinv, r3, bc3, Wl4, bl4, Wl5, bl5, Wl6, bl6)


def kernel(x, edge_index, Wc1, bc1, Wl1, bl1, Wc2, bc2, Wl2, bl2,
           Wc3, bc3, Wl3, bl3, Wl4, bl4, Wl5, bl5, Wl6, bl6):
    edge_index = edge_index.astype(jnp.int32)
    src3 = edge_index[0].reshape(NW, _NCH, _CHUNK)
    dst3 = edge_index[1].reshape(NW, _NCH, _CHUNK)
    ei4 = jnp.stack([src3, dst3], axis=2)
    ones = jnp.ones((_CHUNK, DH), jnp.float32)
    z128 = jnp.zeros((_NR, DH), jnp.float32)

    hist = _sc_hist(dst3, ones, z128)
    t1, r1 = _tc_proj1(x, Wc1, Wl1, bl1.reshape(1, DH))
    hp1, dinv = _tc_scale(t1, hist)
    agg1 = _sc_conv(hp1, ei4, z128, DH)
    hp2, r2 = _tc_combine_proj(agg1, hp1, dinv, r1, bc1.reshape(1, DH),
                               Wc2, Wl2, bl2.reshape(1, DH))
    agg2 = _sc_conv(hp2, ei4, z128, DH)
    Wc3p = jnp.concatenate([Wc3, jnp.zeros((DH - DOUT, DH), jnp.float32)], axis=0)
    hp3, r3 = _tc_combine_proj(agg2, hp2, dinv, r2, bc2.reshape(1, DH),
                               Wc3p, Wl3, bl3.reshape(1, DOUT))
    agg3 = _sc_conv(hp3, ei4, z128, DH)
    imp = _tc_final(agg3, hp3, dinv, r3, bc3.reshape(1, DOUT),
                    Wl4, bl4.reshape(1, DH), Wl5, bl5.reshape(1, DH),
                    Wl6, bl6.reshape(1, DIN))
    return imp
